# Initial kernel scaffold; baseline (speedup 1.0000x reference)
import jax
import jax.numpy as jnp
from jax.experimental import pallas as pl


def kernel(hidden_states, router_w, router_b, gate_w, gate_b, up_w, up_b, down_w, down_b):
    raise NotImplementedError("write your pallas kernel here")


# speedup vs baseline: 1.1486x; 1.1486x over previous
import functools

import jax
import jax.numpy as jnp
from jax import lax
from jax.experimental import pallas as pl
from jax.experimental.pallas import tpu as pltpu
from jax.experimental.pallas import tpu_sc as plsc

E = 8
TOP_K = 2
H = 1024
FF = 2048
ALPHA = 1.702
LIMIT = 7.0
T = 2048

BM = 128
P_PAD = TOP_K * T + E * BM
M_TILES = P_PAD // BM

NC, NS = 2, 16
NW = NC * NS
GCHUNK = 32
H_HALF = H // 2


def _router_body(x_ref, rw_ref, rb_ref, score_ref, idx_ref, p_ref):
    x = x_ref[...]
    logits = lax.dot_general(x, rw_ref[...], (((1,), (1,)), ((), ())),
                             preferred_element_type=jnp.float32) + rb_ref[...]
    iota = lax.broadcasted_iota(jnp.int32, (T, E), 1)
    m1 = jnp.max(logits, axis=1, keepdims=True)
    a1 = jnp.min(jnp.where(logits == m1, iota, E), axis=1, keepdims=True)
    l2 = jnp.where(iota == a1, -jnp.inf, logits)
    m2 = jnp.max(l2, axis=1, keepdims=True)
    a2 = jnp.min(jnp.where(l2 == m2, iota, E), axis=1, keepdims=True)
    ex = jnp.exp(m2 - m1)
    p1 = 1.0 / (1.0 + ex)
    p2 = ex / (1.0 + ex)
    score_ref[...] = jnp.where(iota == a1, p1, 0.0) + jnp.where(iota == a2, p2, 0.0)
    idx_ref[...] = jnp.concatenate([a1, a2], axis=1)
    p_ref[...] = jnp.concatenate([p1, p2], axis=1)


def _router(x, router_w, router_b):
    return pl.pallas_call(
        _router_body,
        out_shape=(
            jax.ShapeDtypeStruct((T, E), jnp.float32),
            jax.ShapeDtypeStruct((T, TOP_K), jnp.int32),
            jax.ShapeDtypeStruct((T, TOP_K), jnp.float32),
        ),
    )(x, router_w, router_b)


def _sc_row_gather(table, idx, n_rows):
    mesh = plsc.VectorSubcoreMesh(core_axis_name="c", subcore_axis_name="s",
                                  num_cores=NC, num_subcores=NS)
    rows_per_w = n_rows // NW

    @functools.partial(
        pl.kernel,
        out_type=jax.ShapeDtypeStruct((n_rows, H), jnp.float32),
        mesh=mesh,
        scratch_types=[
            pltpu.VMEM((GCHUNK,), jnp.int32),
            pltpu.VMEM((GCHUNK, H), jnp.float32),
            pltpu.SemaphoreType.DMA,
        ],
    )
    def k(x_hbm, i_hbm, out_hbm, idx_v, rows_v, sem):
        wid = lax.axis_index("s") * NC + lax.axis_index("c")
        base0 = wid * rows_per_w

        @pl.loop(0, rows_per_w, step=GCHUNK)
        def _(i):
            base = base0 + i
            pltpu.sync_copy(i_hbm.at[pl.ds(base, GCHUNK)], idx_v)
            pltpu.async_copy(x_hbm.at[idx_v], rows_v, sem).wait()
            pltpu.sync_copy(rows_v, out_hbm.at[pl.ds(base, GCHUNK)])

    return k(table, idx)


def _ffn_body(eft_ref, x_ref, w_ref, gw_ref, gb_ref, uw_ref, ub_ref,
              dw_ref, db_ref, o_ref):
    x = x_ref[...]
    gate = lax.dot_general(x, gw_ref[0], (((1,), (1,)), ((), ())),
                           preferred_element_type=jnp.float32) + gb_ref[0]
    gate = jnp.minimum(gate, LIMIT)
    glu = gate * jax.nn.sigmoid(gate * ALPHA)
    up = lax.dot_general(x, uw_ref[0], (((1,), (1,)), ((), ())),
                         preferred_element_type=jnp.float32) + ub_ref[0]
    up = jnp.clip(up, -LIMIT, LIMIT)
    gated = (up + 1.0) * glu
    out = lax.dot_general(gated, dw_ref[0], (((1,), (1,)), ((), ())),
                          preferred_element_type=jnp.float32) + db_ref[0]
    o_ref[...] = out * w_ref[...]


def _ffn(eft, x_sorted, w_sorted, gate_w, gate_b, up_w, up_b, down_w, down_b):
    grid_spec = pltpu.PrefetchScalarGridSpec(
        num_scalar_prefetch=1,
        grid=(M_TILES,),
        in_specs=[
            pl.BlockSpec((BM, H), lambda i, s: (i, 0)),
            pl.BlockSpec((BM, 1), lambda i, s: (i, 0)),
            pl.BlockSpec((1, FF, H), lambda i, s: (s[i], 0, 0)),
            pl.BlockSpec((1, 1, FF), lambda i, s: (s[i], 0, 0)),
            pl.BlockSpec((1, FF, H), lambda i, s: (s[i], 0, 0)),
            pl.BlockSpec((1, 1, FF), lambda i, s: (s[i], 0, 0)),
            pl.BlockSpec((1, H, FF), lambda i, s: (s[i], 0, 0)),
            pl.BlockSpec((1, 1, H), lambda i, s: (s[i], 0, 0)),
        ],
        out_specs=pl.BlockSpec((BM, H), lambda i, s: (i, 0)),
    )
    return pl.pallas_call(
        _ffn_body,
        grid_spec=grid_spec,
        out_shape=jax.ShapeDtypeStruct((P_PAD, H), jnp.float32),
    )(eft, x_sorted, w_sorted, gate_w, gate_b.reshape(E, 1, FF),
      up_w, up_b.reshape(E, 1, FF), down_w, down_b.reshape(E, 1, H))


BN_ADD = 256


def _add_body(a_ref, b_ref, o_ref):
    o_ref[...] = a_ref[...] + b_ref[...]


def _combine_add(g):
    nblk = T // BN_ADD
    return pl.pallas_call(
        _add_body,
        grid=(nblk,),
        in_specs=[
            pl.BlockSpec((BN_ADD, H), lambda i: (i, 0)),
            pl.BlockSpec((BN_ADD, H), lambda i: (i + nblk, 0)),
        ],
        out_specs=pl.BlockSpec((BN_ADD, H), lambda i: (i, 0)),
        out_shape=jax.ShapeDtypeStruct((T, H), jnp.float32),
    )(g, g)


def kernel(hidden_states, router_w, router_b, gate_w, gate_b, up_w, up_b,
           down_w, down_b):
    x = hidden_states.reshape(T, H)
    score, top_idx, top_p = _router(x, router_w, router_b.reshape(1, E))

    pairs = top_idx.reshape(-1)
    onehot = (pairs[:, None] == jnp.arange(E)[None, :]).astype(jnp.int32)
    counts = onehot.sum(0)
    rank = jnp.take_along_axis(jnp.cumsum(onehot, axis=0) - onehot,
                               pairs[:, None], axis=1)[:, 0]
    aligned = ((counts + BM - 1) // BM) * BM
    bounds = jnp.cumsum(aligned)
    off = bounds - aligned
    pos = off[pairs] + rank
    tok_ids = (jnp.arange(TOP_K * T) // TOP_K).astype(jnp.int32)
    tok_sorted = jnp.zeros((P_PAD,), jnp.int32).at[pos].set(tok_ids)
    w_sorted = jnp.zeros((P_PAD,), jnp.float32).at[pos].set(top_p.reshape(-1))
    tile_start = jnp.arange(M_TILES, dtype=jnp.int32) * BM
    eft = jnp.minimum(
        (tile_start[:, None] >= bounds[None, :]).astype(jnp.int32).sum(1), E - 1)

    x_sorted = _sc_row_gather(x, tok_sorted, P_PAD)
    out_sorted = _ffn(eft, x_sorted, w_sorted[:, None], gate_w, gate_b,
                      up_w, up_b, down_w, down_b)
    pos_cat = jnp.concatenate([pos[0::TOP_K], pos[1::TOP_K]]).astype(jnp.int32)
    g = _sc_row_gather(out_sorted, pos_cat, TOP_K * T)
    nxt = _combine_add(g)
    return nxt.reshape(hidden_states.shape), score
